# baseline (speedup 1.0000x reference)
import functools

import jax
import jax.numpy as jnp
from jax.experimental import pallas as pl
from jax.experimental.pallas import tpu as pltpu


def _body(E, R, x_ref, wg_ref, a_ref, b_ref, o_ref):
    x = x_ref[...]
    g = jnp.dot(x, wg_ref[...], preferred_element_type=jnp.float32)
    neg = jnp.float32(-1e30)
    m1 = jnp.max(g, axis=1, keepdims=True)
    is1 = g == m1
    g2 = jnp.where(is1, neg, g)
    m2 = jnp.max(g2, axis=1, keepdims=True)
    is2 = g2 == m2
    t = jnp.exp(m2 - m1)
    w1 = 1.0 / (1.0 + t)
    w2 = t / (1.0 + t)
    wrow = jnp.where(is1, w1, 0.0) + jnp.where(is2, w2, 0.0)

    p = jnp.dot(x, a_ref[...], preferred_element_type=jnp.float32)
    wfull = jnp.repeat(wrow, R, axis=1)
    o_ref[...] = jnp.dot(p * wfull, b_ref[...],
                         preferred_element_type=jnp.float32)


def kernel(inputs, Wg, A, Bm):
    Bsz, S, D = inputs.shape
    E, _, R = A.shape
    T = Bsz * S
    x = inputs.reshape(T, D)
    a_all = jnp.transpose(A, (1, 0, 2)).reshape(D, E * R)
    b_all = Bm.reshape(E * R, D)

    TB = 1024
    out = pl.pallas_call(
        functools.partial(_body, E, R),
        grid=(T // TB,),
        in_specs=[
            pl.BlockSpec((TB, D), lambda i: (i, 0)),
            pl.BlockSpec((D, E), lambda i: (0, 0)),
            pl.BlockSpec((D, E * R), lambda i: (0, 0)),
            pl.BlockSpec((E * R, D), lambda i: (0, 0)),
        ],
        out_specs=pl.BlockSpec((TB, D), lambda i: (i, 0)),
        out_shape=jax.ShapeDtypeStruct((T, D), jnp.float32),
        compiler_params=pltpu.CompilerParams(
            dimension_semantics=("parallel",)),
    )(x, Wg, a_all, b_all)
    return out.reshape(Bsz, S, D)


# speedup vs baseline: 1.2322x; 1.2322x over previous
import functools

import jax
import jax.numpy as jnp
from jax.experimental import pallas as pl
from jax.experimental.pallas import tpu as pltpu


def _body(E, R, x_ref, wg_ref, a_ref, b_ref, e8_ref, o_ref):
    x = x_ref[...]
    g = jnp.dot(x, wg_ref[...], preferred_element_type=jnp.float32)
    neg = jnp.float32(-1e30)
    m1 = jnp.max(g, axis=1, keepdims=True)
    is1 = g == m1
    g2 = jnp.where(is1, neg, g)
    m2 = jnp.max(g2, axis=1, keepdims=True)
    is2 = g2 == m2
    t = jnp.exp(m2 - m1)
    w1 = 1.0 / (1.0 + t)
    w2 = t / (1.0 + t)
    wrow = jnp.where(is1, w1, 0.0) + jnp.where(is2, w2, 0.0)

    p = jnp.dot(x, a_ref[...], preferred_element_type=jnp.float32)
    wfull = jnp.dot(wrow, e8_ref[...], preferred_element_type=jnp.float32)
    o_ref[...] = jnp.dot(p * wfull, b_ref[...],
                         preferred_element_type=jnp.float32)


def kernel(inputs, Wg, A, Bm):
    Bsz, S, D = inputs.shape
    E, _, R = A.shape
    T = Bsz * S
    x = inputs.reshape(T, D)
    a_all = jnp.transpose(A, (1, 0, 2)).reshape(D, E * R)
    b_all = Bm.reshape(E * R, D)
    e8 = (jax.lax.broadcasted_iota(jnp.int32, (E, E * R), 1) // R
          == jax.lax.broadcasted_iota(jnp.int32, (E, E * R), 0)
          ).astype(jnp.float32)

    TB = 1024
    out = pl.pallas_call(
        functools.partial(_body, E, R),
        grid=(T // TB,),
        in_specs=[
            pl.BlockSpec((TB, D), lambda i: (i, 0)),
            pl.BlockSpec((D, E), lambda i: (0, 0)),
            pl.BlockSpec((D, E * R), lambda i: (0, 0)),
            pl.BlockSpec((E * R, D), lambda i: (0, 0)),
            pl.BlockSpec((E, E * R), lambda i: (0, 0)),
        ],
        out_specs=pl.BlockSpec((TB, D), lambda i: (i, 0)),
        out_shape=jax.ShapeDtypeStruct((T, D), jnp.float32),
        compiler_params=pltpu.CompilerParams(
            dimension_semantics=("parallel",)),
    )(x, Wg, a_all, b_all, e8)
    return out.reshape(Bsz, S, D)
